# baseline (speedup 1.0000x reference)
import functools

import jax
import jax.numpy as jnp
from jax import lax
from jax.experimental import pallas as pl
from jax.experimental.pallas import tpu as pltpu
from jax.experimental.pallas import tpu_sc as plsc

N = 100000
E = 3200000
HOPS = 3
HID = 64
GROUPS = [(0, 16), (16, 32), (32, 48), (48, 58)]

RB = 12512

NC, NS, L = 2, 16, 16
NW = NC * NS
NPAD = 100096
SL = NPAD // NS
CR = 8
ROWS_PER_HOP = E // 128
CHUNKS_PER_HOP = ROWS_PER_HOP // CR
TOTAL_CHUNKS = HOPS * CHUNKS_PER_HOP
CPW = (TOTAL_CHUNKS + NW - 1) // NW
TMAX = (CPW + 2) // 3
ZB = 2048


def _mlp_body(x_ref, gw_ref, w1_ref, b1_ref, w2_ref, b2_ref, o_ref):
    xw = x_ref[...] * gw_ref[...]
    h1 = jnp.maximum(jnp.dot(xw, w1_ref[...], preferred_element_type=jnp.float32)
                     + b1_ref[...], 0.0)
    o_ref[...] = jnp.dot(h1, w2_ref[...], preferred_element_type=jnp.float32) \
        + b2_ref[...]


def _combine_body(a_ref, b_ref, c_ref, o_ref):
    o_ref[...] = a_ref[...] + b_ref[...] + c_ref[...]


def _sc_body(h_hbm, ei_hbm, att_hbm, temp_hbm, out0_hbm, out1_hbm,
             *sc):
    (src0, dst0, att0, gat0, src1, dst1, att1, gat1, src2, dst2, att2, gat2,
     zero_v, temp_v, h_vmem, acc_sh,
     sl0, sl1, sl2, ss0, ss1, ss2) = sc
    srcs = (src0, src1, src2)
    dsts = (dst0, dst1, dst2)
    atts = (att0, att1, att2)
    gats = (gat0, gat1, gat2)
    sls = (sl0, sl1, sl2)
    sss = (ss0, ss1, ss2)
    cid = lax.axis_index("c")
    sid = lax.axis_index("s")
    g = cid * NS + sid
    base = sid * SL

    pltpu.sync_copy(h_hbm, h_vmem)

    def _z(i, _):
        zero_v[pl.ds(i * L, L)] = jnp.zeros((L,), jnp.float32)
        return 0
    lax.fori_loop(0, ZB // L, _z, 0)
    pltpu.sync_copy(zero_v, acc_sh.at[pl.ds(base, ZB)])
    pltpu.sync_copy(zero_v, acc_sh.at[pl.ds(base + ZB, ZB)])
    pltpu.sync_copy(zero_v, acc_sh.at[pl.ds(base + 2 * ZB, ZB)])
    pltpu.sync_copy(zero_v.at[pl.ds(0, SL - 3 * ZB)],
                    acc_sh.at[pl.ds(base + 3 * ZB, SL - 3 * ZB)])
    pltpu.sync_copy(temp_hbm, temp_v)
    plsc.subcore_barrier()

    q0 = g * CPW
    cnt = jnp.minimum(CPW, TOTAL_CHUNKS - q0)

    def fire_loads(c, s):
        q = q0 + c
        arow = q * CR
        srow = arow + (q // CHUNKS_PER_HOP) * ROWS_PER_HOP
        pltpu.async_copy(ei_hbm.at[pl.ds(srow, CR), :], srcs[s], sls[s])
        pltpu.async_copy(ei_hbm.at[pl.ds(srow + ROWS_PER_HOP, CR), :],
                         dsts[s], sls[s])
        pltpu.async_copy(att_hbm.at[pl.ds(arow, CR), :], atts[s], sls[s])

    def wait_loads(s):
        pltpu.make_async_copy(ei_hbm.at[pl.ds(0, CR), :], srcs[s], sls[s]).wait()
        pltpu.make_async_copy(ei_hbm.at[pl.ds(0, CR), :], dsts[s], sls[s]).wait()
        pltpu.make_async_copy(att_hbm.at[pl.ds(0, CR), :], atts[s], sls[s]).wait()

    def drain_scatters(s):
        for j in range(CR):
            pltpu.make_async_copy(gats[s].at[j], acc_sh.at[dsts[s].at[j]],
                                  sss[s]).wait()

    def process(c, s):
        wait_loads(s)
        hop = (q0 + c) // CHUNKS_PER_HOP
        t16 = temp_v[hop]

        def _mul(r, _):
            for u in range(8):
                cc = u * L
                idx = srcs[s][r, pl.ds(cc, L)]
                vals = plsc.load_gather(h_vmem, [idx])
                gats[s][r, pl.ds(cc, L)] = vals * atts[s][r, pl.ds(cc, L)] * t16
            return 0
        lax.fori_loop(0, CR, _mul, 0)
        for j in range(CR):
            pltpu.async_copy(gats[s].at[j], acc_sh.at[dsts[s].at[j]], sss[s],
                             add=True)

    @pl.when(cnt > 0)
    def _():
        fire_loads(0, 0)

    def _triple(t, _):
        for u in range(3):
            c = t * 3 + u

            @pl.when(c < cnt)
            def _(c=c, u=u):
                @pl.when(c >= 2)
                def _():
                    drain_scatters((u + 1) % 3)

                @pl.when(c + 1 < cnt)
                def _():
                    fire_loads(c + 1, (u + 1) % 3)

                process(c, u)
        return 0

    lax.fori_loop(0, TMAX, _triple, 0)

    for s in range(3):
        @pl.when((cnt >= 1) & ((cnt - 1) % 3 == s))
        def _(s=s):
            drain_scatters(s)

        @pl.when((cnt >= 2) & ((cnt - 2) % 3 == s))
        def _(s=s):
            drain_scatters(s)

    plsc.subcore_barrier()

    pltpu.sync_copy(acc_sh.at[pl.ds(base, SL)], h_vmem.at[pl.ds(0, SL)])

    @pl.when(cid == 0)
    def _():
        pltpu.sync_copy(h_vmem.at[pl.ds(0, SL)], out0_hbm.at[pl.ds(base, SL)])

    @pl.when(cid == 1)
    def _():
        pltpu.sync_copy(h_vmem.at[pl.ds(0, SL)], out1_hbm.at[pl.ds(base, SL)])


_sc_call = functools.partial(
    pl.kernel,
    mesh=plsc.VectorSubcoreMesh(core_axis_name="c", subcore_axis_name="s"),
    compiler_params=pltpu.CompilerParams(needs_layout_passes=False),
    out_type=(jax.ShapeDtypeStruct((NPAD,), jnp.float32),
              jax.ShapeDtypeStruct((NPAD,), jnp.float32)),
    scratch_types=(
        [pltpu.VMEM((CR, 128), dt)
         for _ in range(3) for dt in (jnp.int32, jnp.int32,
                                      jnp.float32, jnp.float32)]
        + [
            pltpu.VMEM((ZB,), jnp.float32),
            pltpu.VMEM((HOPS, L), jnp.float32),
            pltpu.VMEM((NPAD,), jnp.float32),
            pltpu.VMEM_SHARED((NPAD,), jnp.float32),
        ]
        + [pltpu.SemaphoreType.DMA] * 6
    ),
)(_sc_body)


def kernel(x, hop_edge_index, hop_edge_att, W1, b1, W2, b2, group_weights, temp):
    f32 = jnp.float32
    gw = jnp.concatenate(
        [jnp.full((e - s,), 1.0, f32) * group_weights[i]
         for i, (s, e) in enumerate(GROUPS)])

    h_full = pl.pallas_call(
        _mlp_body,
        grid=(NPAD // RB,),
        in_specs=[
            pl.BlockSpec((RB, 58), lambda i: (i, 0)),
            pl.BlockSpec((1, 58), lambda i: (0, 0)),
            pl.BlockSpec((58, HID), lambda i: (0, 0)),
            pl.BlockSpec((1, HID), lambda i: (0, 0)),
            pl.BlockSpec((HID, 1), lambda i: (0, 0)),
            pl.BlockSpec((1, 1), lambda i: (0, 0)),
        ],
        out_specs=pl.BlockSpec((RB, 1), lambda i: (i, 0)),
        out_shape=jax.ShapeDtypeStruct((NPAD, 1), f32),
    )(x, gw[None, :], W1, b1[None, :], W2, b2[None, :])

    h_flat = h_full.reshape(NPAD)

    ei3 = hop_edge_index.reshape(2 * HOPS * ROWS_PER_HOP, 128)
    att3 = hop_edge_att.reshape(HOPS * ROWS_PER_HOP, 128)
    temp_b = jnp.broadcast_to(temp[:, None], (HOPS, L))

    p0, p1 = _sc_call(h_flat, ei3, att3, temp_b)

    out2d = pl.pallas_call(
        _combine_body,
        out_shape=jax.ShapeDtypeStruct((NPAD // 128, 128), f32),
    )(p0.reshape(NPAD // 128, 128),
      p1.reshape(NPAD // 128, 128),
      h_flat.reshape(NPAD // 128, 128))

    return out2d.reshape(NPAD)[:N].reshape(N, 1)


# speedup vs baseline: 1.9019x; 1.9019x over previous
import functools

import jax
import jax.numpy as jnp
from jax import lax
from jax.experimental import pallas as pl
from jax.experimental.pallas import tpu as pltpu
from jax.experimental.pallas import tpu_sc as plsc

N = 100000
E = 3200000
HOPS = 3
HID = 64
GROUPS = [(0, 16), (16, 32), (32, 48), (48, 58)]

RB = 12512

NC, NS, L = 2, 16, 16
NW = NC * NS
NPAD = 100096
SL = NPAD // NS
CR = 8
ROWS_PER_HOP = E // 128
CHUNKS_PER_HOP = ROWS_PER_HOP // CR
TOTAL_CHUNKS = HOPS * CHUNKS_PER_HOP
CPW = (TOTAL_CHUNKS + NW - 1) // NW
TMAX = (CPW + 2) // 3
BE = CR * 128
ZB = 2048


def _mlp_body(x_ref, gw_ref, w1_ref, b1_ref, w2_ref, b2_ref, o_ref):
    xw = x_ref[...] * gw_ref[...]
    h1 = jnp.maximum(jnp.dot(xw, w1_ref[...], preferred_element_type=jnp.float32)
                     + b1_ref[...], 0.0)
    o_ref[...] = jnp.dot(h1, w2_ref[...], preferred_element_type=jnp.float32) \
        + b2_ref[...]


def _combine_body(a_ref, b_ref, c_ref, o_ref):
    o_ref[...] = a_ref[...] + b_ref[...] + c_ref[...]


def _sc_body(h_hbm, ei_hbm, att_hbm, temp_hbm, out0_hbm, out1_hbm,
             *sc):
    (src0, dst0, att0, gat0, src1, dst1, att1, gat1, src2, dst2, att2, gat2,
     zero_v, temp_v, h_vmem, acc_sh,
     sl0, sl1, sl2, ss0, ss1, ss2) = sc
    srcs = (src0, src1, src2)
    dsts = (dst0, dst1, dst2)
    atts = (att0, att1, att2)
    gats = (gat0, gat1, gat2)
    sls = (sl0, sl1, sl2)
    sss = (ss0, ss1, ss2)
    cid = lax.axis_index("c")
    sid = lax.axis_index("s")
    g = cid * NS + sid
    base = sid * SL

    pltpu.sync_copy(h_hbm, h_vmem)

    def _z(i, _):
        zero_v[pl.ds(i * L, L)] = jnp.zeros((L,), jnp.float32)
        return 0
    lax.fori_loop(0, ZB // L, _z, 0)
    pltpu.sync_copy(zero_v, acc_sh.at[pl.ds(base, ZB)])
    pltpu.sync_copy(zero_v, acc_sh.at[pl.ds(base + ZB, ZB)])
    pltpu.sync_copy(zero_v, acc_sh.at[pl.ds(base + 2 * ZB, ZB)])
    pltpu.sync_copy(zero_v.at[pl.ds(0, SL - 3 * ZB)],
                    acc_sh.at[pl.ds(base + 3 * ZB, SL - 3 * ZB)])
    pltpu.sync_copy(temp_hbm, temp_v)
    plsc.subcore_barrier()

    q0 = g * CPW
    cnt = jnp.minimum(CPW, TOTAL_CHUNKS - q0)

    def fire_loads(c, s):
        q = q0 + c
        hop = q // CHUNKS_PER_HOP
        eoff = (q - hop * CHUNKS_PER_HOP) * BE
        pltpu.async_copy(ei_hbm.at[hop, 0, pl.ds(eoff, BE)], srcs[s], sls[s])
        pltpu.async_copy(ei_hbm.at[hop, 1, pl.ds(eoff, BE)], dsts[s], sls[s])
        pltpu.async_copy(att_hbm.at[hop, pl.ds(eoff, BE)], atts[s], sls[s])

    def wait_loads(s):
        pltpu.make_async_copy(ei_hbm.at[0, 0, pl.ds(0, BE)], srcs[s], sls[s]).wait()
        pltpu.make_async_copy(ei_hbm.at[0, 1, pl.ds(0, BE)], dsts[s], sls[s]).wait()
        pltpu.make_async_copy(att_hbm.at[0, pl.ds(0, BE)], atts[s], sls[s]).wait()

    def drain_scatters(s):
        for j in range(CR):
            pltpu.make_async_copy(gats[s].at[pl.ds(j * 128, 128)],
                                  acc_sh.at[dsts[s].at[pl.ds(j * 128, 128)]],
                                  sss[s]).wait()

    def process(c, s):
        wait_loads(s)
        hop = (q0 + c) // CHUNKS_PER_HOP
        t16 = temp_v[hop]

        def _mul(r, _):
            for u in range(8):
                cc = r * 128 + u * L
                idx = srcs[s][pl.ds(cc, L)]
                vals = plsc.load_gather(h_vmem, [idx])
                gats[s][pl.ds(cc, L)] = vals * atts[s][pl.ds(cc, L)] * t16
            return 0
        lax.fori_loop(0, CR, _mul, 0)
        for j in range(CR):
            pltpu.async_copy(gats[s].at[pl.ds(j * 128, 128)],
                             acc_sh.at[dsts[s].at[pl.ds(j * 128, 128)]], sss[s],
                             add=True)

    @pl.when(cnt > 0)
    def _():
        fire_loads(0, 0)

    def _triple(t, _):
        for u in range(3):
            c = t * 3 + u

            @pl.when(c < cnt)
            def _(c=c, u=u):
                @pl.when(c >= 2)
                def _():
                    drain_scatters((u + 1) % 3)

                @pl.when(c + 1 < cnt)
                def _():
                    fire_loads(c + 1, (u + 1) % 3)

                process(c, u)
        return 0

    lax.fori_loop(0, TMAX, _triple, 0)

    for s in range(3):
        @pl.when((cnt >= 1) & ((cnt - 1) % 3 == s))
        def _(s=s):
            drain_scatters(s)

        @pl.when((cnt >= 2) & ((cnt - 2) % 3 == s))
        def _(s=s):
            drain_scatters(s)

    plsc.subcore_barrier()

    pltpu.sync_copy(acc_sh.at[pl.ds(base, SL)], h_vmem.at[pl.ds(0, SL)])

    @pl.when(cid == 0)
    def _():
        pltpu.sync_copy(h_vmem.at[pl.ds(0, SL)], out0_hbm.at[pl.ds(base, SL)])

    @pl.when(cid == 1)
    def _():
        pltpu.sync_copy(h_vmem.at[pl.ds(0, SL)], out1_hbm.at[pl.ds(base, SL)])


_sc_call = functools.partial(
    pl.kernel,
    mesh=plsc.VectorSubcoreMesh(core_axis_name="c", subcore_axis_name="s"),
    compiler_params=pltpu.CompilerParams(needs_layout_passes=False),
    out_type=(jax.ShapeDtypeStruct((NPAD,), jnp.float32),
              jax.ShapeDtypeStruct((NPAD,), jnp.float32)),
    scratch_types=(
        [pltpu.VMEM((BE,), dt)
         for _ in range(3) for dt in (jnp.int32, jnp.int32,
                                      jnp.float32, jnp.float32)]
        + [
            pltpu.VMEM((ZB,), jnp.float32),
            pltpu.VMEM((HOPS, L), jnp.float32),
            pltpu.VMEM((NPAD,), jnp.float32),
            pltpu.VMEM_SHARED((NPAD,), jnp.float32),
        ]
        + [pltpu.SemaphoreType.DMA] * 6
    ),
)(_sc_body)


def kernel(x, hop_edge_index, hop_edge_att, W1, b1, W2, b2, group_weights, temp):
    f32 = jnp.float32
    gw = jnp.concatenate(
        [jnp.full((e - s,), 1.0, f32) * group_weights[i]
         for i, (s, e) in enumerate(GROUPS)])

    h_full = pl.pallas_call(
        _mlp_body,
        grid=(NPAD // RB,),
        in_specs=[
            pl.BlockSpec((RB, 58), lambda i: (i, 0)),
            pl.BlockSpec((1, 58), lambda i: (0, 0)),
            pl.BlockSpec((58, HID), lambda i: (0, 0)),
            pl.BlockSpec((1, HID), lambda i: (0, 0)),
            pl.BlockSpec((HID, 1), lambda i: (0, 0)),
            pl.BlockSpec((1, 1), lambda i: (0, 0)),
        ],
        out_specs=pl.BlockSpec((RB, 1), lambda i: (i, 0)),
        out_shape=jax.ShapeDtypeStruct((NPAD, 1), f32),
    )(x, gw[None, :], W1, b1[None, :], W2, b2[None, :])

    h_flat = h_full.reshape(NPAD)

    temp_b = jnp.broadcast_to(temp[:, None], (HOPS, L))

    p0, p1 = _sc_call(h_flat, hop_edge_index, hop_edge_att, temp_b)

    out2d = pl.pallas_call(
        _combine_body,
        out_shape=jax.ShapeDtypeStruct((NPAD // 128, 128), f32),
    )(p0.reshape(NPAD // 128, 128),
      p1.reshape(NPAD // 128, 128),
      h_flat.reshape(NPAD // 128, 128))

    return out2d.reshape(NPAD)[:N].reshape(N, 1)


# speedup vs baseline: 2.3985x; 1.2611x over previous
import functools

import jax
import jax.numpy as jnp
from jax import lax
from jax.experimental import pallas as pl
from jax.experimental.pallas import tpu as pltpu
from jax.experimental.pallas import tpu_sc as plsc

N = 100000
E = 3200000
HOPS = 3
HID = 64
GROUPS = [(0, 16), (16, 32), (32, 48), (48, 58)]

CB = 5888

NC, NS, L = 2, 16, 16
NW = NC * NS
NPAD = 100096
SL = NPAD // NS
CR = 8
ROWS_PER_HOP = E // 128
CHUNKS_PER_HOP = ROWS_PER_HOP // CR
TOTAL_CHUNKS = HOPS * CHUNKS_PER_HOP
CPW = (TOTAL_CHUNKS + NW - 1) // NW
TMAX = (CPW + 2) // 3
BE = CR * 128
ZB = 2048


def _mlp_body(xt_ref, gw_ref, w1_ref, b1_ref, w2_ref, b2_ref, o_ref):
    xw = xt_ref[...] * gw_ref[...]
    z = lax.dot_general(w1_ref[...], xw, (((0,), (0,)), ((), ())),
                        preferred_element_type=jnp.float32)
    h1 = jnp.maximum(z + b1_ref[...], 0.0)
    o_ref[...] = lax.dot_general(w2_ref[...], h1, (((0,), (0,)), ((), ())),
                                 preferred_element_type=jnp.float32) \
        + b2_ref[...]


def _combine_body(a_ref, b_ref, c_ref, o_ref):
    o_ref[...] = a_ref[...] + b_ref[...] + c_ref[...]


def _sc_body(h_hbm, ei_hbm, att_hbm, temp_hbm, out0_hbm, out1_hbm,
             *sc):
    (src0, dst0, att0, gat0, src1, dst1, att1, gat1, src2, dst2, att2, gat2,
     zero_v, temp_v, h_vmem, acc_sh,
     sl0, sl1, sl2, ss0, ss1, ss2) = sc
    srcs = (src0, src1, src2)
    dsts = (dst0, dst1, dst2)
    atts = (att0, att1, att2)
    gats = (gat0, gat1, gat2)
    sls = (sl0, sl1, sl2)
    sss = (ss0, ss1, ss2)
    cid = lax.axis_index("c")
    sid = lax.axis_index("s")
    g = cid * NS + sid
    base = sid * SL

    pltpu.sync_copy(h_hbm, h_vmem)

    def _z(i, _):
        zero_v[pl.ds(i * L, L)] = jnp.zeros((L,), jnp.float32)
        return 0
    lax.fori_loop(0, ZB // L, _z, 0)
    pltpu.sync_copy(zero_v, acc_sh.at[pl.ds(base, ZB)])
    pltpu.sync_copy(zero_v, acc_sh.at[pl.ds(base + ZB, ZB)])
    pltpu.sync_copy(zero_v, acc_sh.at[pl.ds(base + 2 * ZB, ZB)])
    pltpu.sync_copy(zero_v.at[pl.ds(0, SL - 3 * ZB)],
                    acc_sh.at[pl.ds(base + 3 * ZB, SL - 3 * ZB)])
    pltpu.sync_copy(temp_hbm, temp_v)
    plsc.subcore_barrier()

    q0 = g * CPW
    cnt = jnp.minimum(CPW, TOTAL_CHUNKS - q0)

    def fire_loads(c, s):
        q = q0 + c
        hop = q // CHUNKS_PER_HOP
        eoff = (q - hop * CHUNKS_PER_HOP) * BE
        pltpu.async_copy(ei_hbm.at[hop, 0, pl.ds(eoff, BE)], srcs[s], sls[s])
        pltpu.async_copy(ei_hbm.at[hop, 1, pl.ds(eoff, BE)], dsts[s], sls[s])
        pltpu.async_copy(att_hbm.at[hop, pl.ds(eoff, BE)], atts[s], sls[s])

    def wait_loads(s):
        pltpu.make_async_copy(ei_hbm.at[0, 0, pl.ds(0, BE)], srcs[s], sls[s]).wait()
        pltpu.make_async_copy(ei_hbm.at[0, 1, pl.ds(0, BE)], dsts[s], sls[s]).wait()
        pltpu.make_async_copy(att_hbm.at[0, pl.ds(0, BE)], atts[s], sls[s]).wait()

    def drain_scatters(s):
        for j in range(CR):
            pltpu.make_async_copy(gats[s].at[pl.ds(j * 128, 128)],
                                  acc_sh.at[dsts[s].at[pl.ds(j * 128, 128)]],
                                  sss[s]).wait()

    def process(c, s):
        wait_loads(s)
        hop = (q0 + c) // CHUNKS_PER_HOP
        t16 = temp_v[hop]

        def _mul(r, _):
            for u in range(8):
                cc = r * 128 + u * L
                idx = srcs[s][pl.ds(cc, L)]
                vals = plsc.load_gather(h_vmem, [idx])
                gats[s][pl.ds(cc, L)] = vals * atts[s][pl.ds(cc, L)] * t16
            return 0
        lax.fori_loop(0, CR, _mul, 0)
        for j in range(CR):
            pltpu.async_copy(gats[s].at[pl.ds(j * 128, 128)],
                             acc_sh.at[dsts[s].at[pl.ds(j * 128, 128)]], sss[s],
                             add=True)

    @pl.when(cnt > 0)
    def _():
        fire_loads(0, 0)

    def _triple(t, _):
        for u in range(3):
            c = t * 3 + u

            @pl.when(c < cnt)
            def _(c=c, u=u):
                @pl.when(c >= 2)
                def _():
                    drain_scatters((u + 1) % 3)

                @pl.when(c + 1 < cnt)
                def _():
                    fire_loads(c + 1, (u + 1) % 3)

                process(c, u)
        return 0

    lax.fori_loop(0, TMAX, _triple, 0)

    for s in range(3):
        @pl.when((cnt >= 1) & ((cnt - 1) % 3 == s))
        def _(s=s):
            drain_scatters(s)

        @pl.when((cnt >= 2) & ((cnt - 2) % 3 == s))
        def _(s=s):
            drain_scatters(s)

    plsc.subcore_barrier()

    pltpu.sync_copy(acc_sh.at[pl.ds(base, SL)], h_vmem.at[pl.ds(0, SL)])

    @pl.when(cid == 0)
    def _():
        pltpu.sync_copy(h_vmem.at[pl.ds(0, SL)], out0_hbm.at[pl.ds(base, SL)])

    @pl.when(cid == 1)
    def _():
        pltpu.sync_copy(h_vmem.at[pl.ds(0, SL)], out1_hbm.at[pl.ds(base, SL)])


_sc_call = functools.partial(
    pl.kernel,
    mesh=plsc.VectorSubcoreMesh(core_axis_name="c", subcore_axis_name="s"),
    compiler_params=pltpu.CompilerParams(needs_layout_passes=False),
    out_type=(jax.ShapeDtypeStruct((NPAD,), jnp.float32),
              jax.ShapeDtypeStruct((NPAD,), jnp.float32)),
    scratch_types=(
        [pltpu.VMEM((BE,), dt)
         for _ in range(3) for dt in (jnp.int32, jnp.int32,
                                      jnp.float32, jnp.float32)]
        + [
            pltpu.VMEM((ZB,), jnp.float32),
            pltpu.VMEM((HOPS, L), jnp.float32),
            pltpu.VMEM((NPAD,), jnp.float32),
            pltpu.VMEM_SHARED((NPAD,), jnp.float32),
        ]
        + [pltpu.SemaphoreType.DMA] * 6
    ),
)(_sc_body)


def kernel(x, hop_edge_index, hop_edge_att, W1, b1, W2, b2, group_weights, temp):
    f32 = jnp.float32
    gw = jnp.concatenate(
        [jnp.full((e - s,), 1.0, f32) * group_weights[i]
         for i, (s, e) in enumerate(GROUPS)])

    h_row = pl.pallas_call(
        _mlp_body,
        grid=(NPAD // CB,),
        in_specs=[
            pl.BlockSpec((58, CB), lambda i: (0, i)),
            pl.BlockSpec((58, 1), lambda i: (0, 0)),
            pl.BlockSpec((58, HID), lambda i: (0, 0)),
            pl.BlockSpec((HID, 1), lambda i: (0, 0)),
            pl.BlockSpec((HID, 1), lambda i: (0, 0)),
            pl.BlockSpec((1, 1), lambda i: (0, 0)),
        ],
        out_specs=pl.BlockSpec((1, CB), lambda i: (0, i)),
        out_shape=jax.ShapeDtypeStruct((1, NPAD), f32),
    )(x.T, gw[:, None], W1, b1[:, None], W2, b2[None, :])

    h_flat = h_row.reshape(NPAD)

    temp_b = jnp.broadcast_to(temp[:, None], (HOPS, L))

    p0, p1 = _sc_call(h_flat, hop_edge_index, hop_edge_att, temp_b)

    out2d = pl.pallas_call(
        _combine_body,
        out_shape=jax.ShapeDtypeStruct((NPAD // 128, 128), f32),
    )(p0.reshape(NPAD // 128, 128),
      p1.reshape(NPAD // 128, 128),
      h_flat.reshape(NPAD // 128, 128))

    return out2d.reshape(NPAD)[:N].reshape(N, 1)
